# initial kernel scaffold (unmeasured)
import functools

import jax
import jax.numpy as jnp
from jax import lax
from jax.experimental import pallas as pl
from jax.experimental.pallas import tpu as pltpu

N_DEV = 8


def kernel(x, w_mat, scale_x, scale_w):
    m, k_local = x.shape
    _, n = w_mat.shape
    ch = m // N_DEV

    def body(x_ref, w_ref, sx_ref, sw_ref, out_ref,
             xb_ref, wb_ref, send_ref, recv_ref,
             send_sem, recv_sem, credit_sem):
        my = lax.axis_index("i")
        left = lax.rem(my + N_DEV - 1, N_DEV)
        right = lax.rem(my + 1, N_DEV)

        scale = sx_ref[0] * sw_ref[0]

        xb_ref[:, :] = x_ref[:, :].astype(jnp.bfloat16)
        wb_ref[:, :] = w_ref[:, :].astype(jnp.bfloat16)

        def partial_chunk(c):
            xs = xb_ref[pl.ds(c * ch, ch), :]
            return jax.lax.dot_general(
                xs, wb_ref[:, :],
                dimension_numbers=(((1,), (0,)), ((), ())),
                preferred_element_type=jnp.float32,
            )

        def store_out(c, acc):
            out_ref[pl.ds(c * ch, ch), :] = jnp.maximum(acc * scale, 0.0)

        def signal_credit():
            pl.semaphore_signal(
                credit_sem, inc=1,
                device_id=(left,), device_id_type=pl.DeviceIdType.MESH,
            )

        def hop():
            rdma = pltpu.make_async_remote_copy(
                src_ref=send_ref, dst_ref=recv_ref,
                send_sem=send_sem, recv_sem=recv_sem,
                device_id=(right,), device_id_type=pl.DeviceIdType.MESH,
            )
            rdma.start()
            rdma.wait()

        barrier = pltpu.get_barrier_semaphore()
        for nbr in (left, right):
            pl.semaphore_signal(
                barrier, inc=1,
                device_id=(nbr,), device_id_type=pl.DeviceIdType.MESH,
            )
        pl.semaphore_wait(barrier, 2)

        for s in range(N_DEV - 1):
            c_send = lax.rem(my + N_DEV - s, N_DEV)
            if s == 0:
                send_ref[:, :] = partial_chunk(c_send)
            else:
                send_ref[:, :] = recv_ref[:, :] + partial_chunk(c_send)
                signal_credit()
                pl.semaphore_wait(credit_sem, 1)
            hop()

        c_mine = lax.rem(my + 1, N_DEV)
        acc = recv_ref[:, :] + partial_chunk(c_mine)
        signal_credit()
        send_ref[:, :] = acc
        store_out(c_mine, acc)

        for t in range(N_DEV - 1):
            if t > 0:
                send_ref[:, :] = recv_ref[:, :]
                signal_credit()
            pl.semaphore_wait(credit_sem, 1)
            hop()
            store_out(lax.rem(my + N_DEV - t, N_DEV), recv_ref[:, :])

        @functools.partial(pl.run_scoped, sem=pltpu.SemaphoreType.REGULAR)
        def _(sem):
            for nbr in (left, right):
                pl.semaphore_signal(
                    sem, inc=1,
                    device_id=(nbr,), device_id_type=pl.DeviceIdType.MESH,
                )
            pl.semaphore_wait(sem, 2)

    return pl.pallas_call(
        body,
        out_shape=jax.ShapeDtypeStruct((m, n), jnp.float32),
        in_specs=[
            pl.BlockSpec(memory_space=pltpu.VMEM),
            pl.BlockSpec(memory_space=pltpu.VMEM),
            pl.BlockSpec(memory_space=pltpu.SMEM),
            pl.BlockSpec(memory_space=pltpu.SMEM),
        ],
        out_specs=pl.BlockSpec(memory_space=pltpu.VMEM),
        scratch_shapes=[
            pltpu.VMEM((m, k_local), jnp.bfloat16),
            pltpu.VMEM((k_local, n), jnp.bfloat16),
            pltpu.VMEM((ch, n), jnp.float32),
            pltpu.VMEM((ch, n), jnp.float32),
            pltpu.SemaphoreType.DMA,
            pltpu.SemaphoreType.DMA,
            pltpu.SemaphoreType.REGULAR,
        ],
        compiler_params=pltpu.CompilerParams(collective_id=0),
    )(x, w_mat, scale_x, scale_w)


# baseline (device time: 729972 ns/iter reference)
import functools

import jax
import jax.numpy as jnp
from jax import lax
from jax.experimental import pallas as pl
from jax.experimental.pallas import tpu as pltpu

N_DEV = 8


def kernel(x, w_mat, scale_x, scale_w):
    m, k_local = x.shape
    _, n = w_mat.shape
    ch = m // N_DEV

    def body(x_ref, w_ref, sx_ref, sw_ref, out_ref,
             xb_ref, wb_ref, send_ref, recv_ref,
             send_sem, recv_sem, credit_sem):
        my = lax.axis_index("i")
        left = lax.rem(my + N_DEV - 1, N_DEV)
        right = lax.rem(my + 1, N_DEV)

        scale = sx_ref[0] * sw_ref[0]

        xb_ref[:, :] = x_ref[:, :].astype(jnp.bfloat16)
        wb_ref[:, :] = w_ref[:, :].astype(jnp.bfloat16)

        def partial_chunk(c):
            xs = xb_ref[pl.ds(c * ch, ch), :]
            return jax.lax.dot_general(
                xs, wb_ref[:, :],
                dimension_numbers=(((1,), (0,)), ((), ())),
                preferred_element_type=jnp.float32,
            )

        def store_out(c, acc):
            out_ref[pl.ds(c * ch, ch), :] = jnp.maximum(acc * scale, 0.0)

        def signal_credit():
            pl.semaphore_signal(
                credit_sem, inc=1,
                device_id=(left,), device_id_type=pl.DeviceIdType.MESH,
            )

        def hop():
            rdma = pltpu.make_async_remote_copy(
                src_ref=send_ref, dst_ref=recv_ref,
                send_sem=send_sem, recv_sem=recv_sem,
                device_id=(right,), device_id_type=pl.DeviceIdType.MESH,
            )
            rdma.start()
            rdma.wait()

        barrier = pltpu.get_barrier_semaphore()
        for nbr in (left, right):
            pl.semaphore_signal(
                barrier, inc=1,
                device_id=(nbr,), device_id_type=pl.DeviceIdType.MESH,
            )
        pl.semaphore_wait(barrier, 2)

        for s in range(N_DEV - 1):
            c_send = lax.rem(my + N_DEV - s, N_DEV)
            if s == 0:
                send_ref[:, :] = partial_chunk(c_send)
            else:
                send_ref[:, :] = recv_ref[:, :] + partial_chunk(c_send)
                signal_credit()
                pl.semaphore_wait(credit_sem, 1)
            hop()

        c_mine = lax.rem(my + 1, N_DEV)
        acc = recv_ref[:, :] + partial_chunk(c_mine)
        signal_credit()
        send_ref[:, :] = acc
        store_out(c_mine, acc)

        for t in range(N_DEV - 1):
            if t > 0:
                send_ref[:, :] = recv_ref[:, :]
                signal_credit()
            pl.semaphore_wait(credit_sem, 1)
            hop()
            store_out(lax.rem(my + N_DEV - t, N_DEV), recv_ref[:, :])

        @functools.partial(pl.run_scoped, sem=pltpu.SemaphoreType.REGULAR)
        def _(sem):
            for nbr in (left, right):
                pl.semaphore_signal(
                    sem, inc=1,
                    device_id=(nbr,), device_id_type=pl.DeviceIdType.MESH,
                )
            pl.semaphore_wait(sem, 2)

    return pl.pallas_call(
        body,
        out_shape=jax.ShapeDtypeStruct((m, n), jnp.float32),
        in_specs=[
            pl.BlockSpec(memory_space=pltpu.VMEM),
            pl.BlockSpec(memory_space=pltpu.VMEM),
            pl.BlockSpec(memory_space=pltpu.SMEM),
            pl.BlockSpec(memory_space=pltpu.SMEM),
        ],
        out_specs=pl.BlockSpec(memory_space=pltpu.VMEM),
        scratch_shapes=[
            pltpu.VMEM((m, k_local), jnp.bfloat16),
            pltpu.VMEM((k_local, n), jnp.bfloat16),
            pltpu.VMEM((ch, n), jnp.float32),
            pltpu.VMEM((ch, n), jnp.float32),
            pltpu.SemaphoreType.DMA,
            pltpu.SemaphoreType.DMA,
            pltpu.SemaphoreType.REGULAR,
        ],
        compiler_params=pltpu.CompilerParams(
            collective_id=0,
            vmem_limit_bytes=64 * 1024 * 1024,
        ),
    )(x, w_mat, scale_x, scale_w)


# device time: 417815 ns/iter; 1.7471x vs baseline; 1.7471x over previous
import functools

import jax
import jax.numpy as jnp
from jax import lax
from jax.experimental import pallas as pl
from jax.experimental.pallas import tpu as pltpu

N_DEV = 8


def kernel(x, w_mat, scale_x, scale_w):
    m, k_local = x.shape
    _, n = w_mat.shape
    ch = m // N_DEV
    hn = n // 2

    def body(x_ref, w_ref, sx_ref, sw_ref, out_ref,
             xb_ref, wb_ref,
             send_cw, recv_cw, send_ccw, recv_ccw,
             send_sems, recv_sems, credit_sems):
        my = lax.axis_index("i")
        left = lax.rem(my + N_DEV - 1, N_DEV)
        right = lax.rem(my + 1, N_DEV)

        scale = sx_ref[0] * sw_ref[0]

        xb_ref[:, :] = x_ref[:, :].astype(jnp.bfloat16)
        wb_ref[:, :] = w_ref[:, :].astype(jnp.bfloat16)

        def partial_chunk(c, half):
            xs = xb_ref[pl.ds(c * ch, ch), :]
            return jax.lax.dot_general(
                xs, wb_ref[:, half * hn:(half + 1) * hn],
                dimension_numbers=(((1,), (0,)), ((), ())),
                preferred_element_type=jnp.float32,
            )

        def store_out(c, half, acc):
            out_ref[pl.ds(c * ch, ch), half * hn:(half + 1) * hn] = (
                jnp.maximum(acc * scale, 0.0))

        send_peer = (right, left)
        recv_peer = (left, right)
        send_bufs = (send_cw, send_ccw)
        recv_bufs = (recv_cw, recv_ccw)

        def c_rs(d, s):
            off = (N_DEV - s) if d == 0 else s
            return lax.rem(my + off, N_DEV)

        def c_ag(d, t):
            off = (N_DEV - t) if d == 0 else t
            return lax.rem(my + off, N_DEV)

        def signal_credit(d):
            pl.semaphore_signal(
                credit_sems.at[d], inc=1,
                device_id=(recv_peer[d],),
                device_id_type=pl.DeviceIdType.MESH,
            )

        def start_hop(d):
            rdma = pltpu.make_async_remote_copy(
                src_ref=send_bufs[d], dst_ref=recv_bufs[d],
                send_sem=send_sems.at[d], recv_sem=recv_sems.at[d],
                device_id=(send_peer[d],),
                device_id_type=pl.DeviceIdType.MESH,
            )
            rdma.start()
            return rdma

        barrier = pltpu.get_barrier_semaphore()
        for nbr in (left, right):
            pl.semaphore_signal(
                barrier, inc=1,
                device_id=(nbr,), device_id_type=pl.DeviceIdType.MESH,
            )
        pl.semaphore_wait(barrier, 2)

        for s in range(N_DEV - 1):
            if s == 0:
                for d in (0, 1):
                    send_bufs[d][:, :] = partial_chunk(c_rs(d, 0), d)
            else:
                for d in (0, 1):
                    send_bufs[d][:, :] = (
                        recv_bufs[d][:, :] + partial_chunk(c_rs(d, s), d))
                    signal_credit(d)
                for d in (0, 1):
                    pl.semaphore_wait(credit_sems.at[d], 1)
            rdmas = [start_hop(0), start_hop(1)]
            for r in rdmas:
                r.wait()

        for d in (0, 1):
            c_fin = lax.rem(my + (1 if d == 0 else N_DEV - 1), N_DEV)
            acc = recv_bufs[d][:, :] + partial_chunk(c_fin, d)
            signal_credit(d)
            send_bufs[d][:, :] = acc
            store_out(c_fin, d, acc)

        for t in range(N_DEV - 1):
            if t > 0:
                for d in (0, 1):
                    send_bufs[d][:, :] = recv_bufs[d][:, :]
                    signal_credit(d)
            for d in (0, 1):
                pl.semaphore_wait(credit_sems.at[d], 1)
            rdmas = [start_hop(0), start_hop(1)]
            for r in rdmas:
                r.wait()
            for d in (0, 1):
                store_out(c_ag(d, t), d, recv_bufs[d][:, :])

        @functools.partial(pl.run_scoped, sem=pltpu.SemaphoreType.REGULAR)
        def _(sem):
            for nbr in (left, right):
                pl.semaphore_signal(
                    sem, inc=1,
                    device_id=(nbr,), device_id_type=pl.DeviceIdType.MESH,
                )
            pl.semaphore_wait(sem, 2)

    return pl.pallas_call(
        body,
        out_shape=jax.ShapeDtypeStruct((m, n), jnp.float32),
        in_specs=[
            pl.BlockSpec(memory_space=pltpu.VMEM),
            pl.BlockSpec(memory_space=pltpu.VMEM),
            pl.BlockSpec(memory_space=pltpu.SMEM),
            pl.BlockSpec(memory_space=pltpu.SMEM),
        ],
        out_specs=pl.BlockSpec(memory_space=pltpu.VMEM),
        scratch_shapes=[
            pltpu.VMEM((m, k_local), jnp.bfloat16),
            pltpu.VMEM((k_local, n), jnp.bfloat16),
            pltpu.VMEM((ch, hn), jnp.float32),
            pltpu.VMEM((ch, hn), jnp.float32),
            pltpu.VMEM((ch, hn), jnp.float32),
            pltpu.VMEM((ch, hn), jnp.float32),
            pltpu.SemaphoreType.DMA((2,)),
            pltpu.SemaphoreType.DMA((2,)),
            pltpu.SemaphoreType.REGULAR((2,)),
        ],
        compiler_params=pltpu.CompilerParams(
            collective_id=0,
            vmem_limit_bytes=64 * 1024 * 1024,
        ),
    )(x, w_mat, scale_x, scale_w)


# device time: 366399 ns/iter; 1.9923x vs baseline; 1.1403x over previous
import functools

import jax
import jax.numpy as jnp
from jax import lax
from jax.experimental import pallas as pl
from jax.experimental.pallas import tpu as pltpu

N_DEV = 8
N_RING = 4


def kernel(x, w_mat, scale_x, scale_w):
    m, k_local = x.shape
    _, n = w_mat.shape
    ch = m // N_DEV
    qn = n // N_RING

    def body(x_ref, w_ref, sx_ref, sw_ref, out_ref,
             xb_ref, wb_ref, send_bufs, recv_bufs,
             send_sems, recv_sems, credit_sems):
        my = lax.axis_index("i")
        left = lax.rem(my + N_DEV - 1, N_DEV)
        right = lax.rem(my + 1, N_DEV)

        scale = sx_ref[0] * sw_ref[0]

        xb_ref[:, :] = x_ref[:, :].astype(jnp.bfloat16)
        wb_ref[:, :] = w_ref[:, :].astype(jnp.bfloat16)

        cw = lambda r: r < 2
        send_peer = lambda r: right if cw(r) else left
        recv_peer = lambda r: left if cw(r) else right

        def pc(c, r):
            xs = xb_ref[pl.ds(c * ch, ch), :]
            return jax.lax.dot_general(
                xs, wb_ref[:, r * qn:(r + 1) * qn],
                dimension_numbers=(((1,), (0,)), ((), ())),
                preferred_element_type=jnp.float32,
            )

        def c_rs(r, s):
            return lax.rem(my + (N_DEV - s if cw(r) else s), N_DEV)

        def c_fin(r):
            return lax.rem(my + (1 if cw(r) else N_DEV - 1), N_DEV)

        def c_ag(r, k):
            return lax.rem(my + (N_DEV - k if cw(r) else k), N_DEV)

        def store_out(c, r, acc):
            out_ref[pl.ds(c * ch, ch), r * qn:(r + 1) * qn] = (
                jnp.maximum(acc * scale, 0.0))

        def signal_credit(r):
            pl.semaphore_signal(
                credit_sems.at[r], inc=1,
                device_id=(recv_peer(r),),
                device_id_type=pl.DeviceIdType.MESH,
            )

        def start_hop(r):
            rdma = pltpu.make_async_remote_copy(
                src_ref=send_bufs.at[r], dst_ref=recv_bufs.at[r],
                send_sem=send_sems.at[r], recv_sem=recv_sems.at[r],
                device_id=(send_peer(r),),
                device_id_type=pl.DeviceIdType.MESH,
            )
            rdma.start()
            return rdma

        barrier = pltpu.get_barrier_semaphore()
        for nbr in (left, right):
            pl.semaphore_signal(
                barrier, inc=1,
                device_id=(nbr,), device_id_type=pl.DeviceIdType.MESH,
            )
        pl.semaphore_wait(barrier, 2)

        rings = range(N_RING)
        rdmas = [None] * N_RING

        for r in rings:
            send_bufs[r, :, :] = pc(c_rs(r, 0), r)
        for r in rings:
            rdmas[r] = start_hop(r)

        for s in range(1, N_DEV - 1):
            for r in rings:
                rdmas[r].wait()
                send_bufs[r, :, :] = recv_bufs[r, :, :] + pc(c_rs(r, s), r)
                signal_credit(r)
                pl.semaphore_wait(credit_sems.at[r], 1)
                rdmas[r] = start_hop(r)

        for r in rings:
            rdmas[r].wait()
            acc = recv_bufs[r, :, :] + pc(c_fin(r), r)
            signal_credit(r)
            send_bufs[r, :, :] = acc
            pl.semaphore_wait(credit_sems.at[r], 1)
            rdmas[r] = start_hop(r)
            store_out(c_fin(r), r, acc)

        for t in range(1, N_DEV - 1):
            for r in rings:
                rdmas[r].wait()
                send_bufs[r, :, :] = recv_bufs[r, :, :]
                signal_credit(r)
                pl.semaphore_wait(credit_sems.at[r], 1)
                rdmas[r] = start_hop(r)
                store_out(c_ag(r, t - 1), r, send_bufs[r, :, :])

        for r in rings:
            rdmas[r].wait()
            store_out(c_ag(r, N_DEV - 2), r, recv_bufs[r, :, :])

        @functools.partial(pl.run_scoped, sem=pltpu.SemaphoreType.REGULAR)
        def _(sem):
            for nbr in (left, right):
                pl.semaphore_signal(
                    sem, inc=1,
                    device_id=(nbr,), device_id_type=pl.DeviceIdType.MESH,
                )
            pl.semaphore_wait(sem, 2)

    return pl.pallas_call(
        body,
        out_shape=jax.ShapeDtypeStruct((m, n), jnp.float32),
        in_specs=[
            pl.BlockSpec(memory_space=pltpu.VMEM),
            pl.BlockSpec(memory_space=pltpu.VMEM),
            pl.BlockSpec(memory_space=pltpu.SMEM),
            pl.BlockSpec(memory_space=pltpu.SMEM),
        ],
        out_specs=pl.BlockSpec(memory_space=pltpu.VMEM),
        scratch_shapes=[
            pltpu.VMEM((m, k_local), jnp.bfloat16),
            pltpu.VMEM((k_local, n), jnp.bfloat16),
            pltpu.VMEM((N_RING, ch, qn), jnp.float32),
            pltpu.VMEM((N_RING, ch, qn), jnp.float32),
            pltpu.SemaphoreType.DMA((N_RING,)),
            pltpu.SemaphoreType.DMA((N_RING,)),
            pltpu.SemaphoreType.REGULAR((N_RING,)),
        ],
        compiler_params=pltpu.CompilerParams(
            collective_id=0,
            vmem_limit_bytes=64 * 1024 * 1024,
        ),
    )(x, w_mat, scale_x, scale_w)


# device time: 206500 ns/iter; 3.5350x vs baseline; 1.7743x over previous
import functools

import jax
import jax.numpy as jnp
from jax import lax
from jax.experimental import pallas as pl
from jax.experimental.pallas import tpu as pltpu

N_DEV = 8
N_RING = 4


def kernel(x, w_mat, scale_x, scale_w):
    m, k_local = x.shape
    _, n = w_mat.shape
    ch = m // N_DEV
    qn = n // N_RING

    def body(x_ref, w_ref, sx_ref, sw_ref, out_ref,
             xb_ref, wb_ref, send_bufs, recv_bufs,
             send_sems, recv_sems, credit_sems):
        my = lax.axis_index("i")
        left = lax.rem(my + N_DEV - 1, N_DEV)
        right = lax.rem(my + 1, N_DEV)

        scale = sx_ref[0] * sw_ref[0]

        xb_ref[:, :] = x_ref[:, :].astype(jnp.bfloat16)
        wb_ref[:, :] = w_ref[:, :].astype(jnp.bfloat16)

        cw = lambda r: r < 2
        send_peer = lambda r: right if cw(r) else left
        recv_peer = lambda r: left if cw(r) else right

        def pc(c, r):
            xs = xb_ref[pl.ds(c * ch, ch), :]
            return jax.lax.dot_general(
                xs, wb_ref[:, r * qn:(r + 1) * qn],
                dimension_numbers=(((1,), (0,)), ((), ())),
                preferred_element_type=jnp.float32,
            )

        def c_rs(r, s):
            return lax.rem(my + (N_DEV - s if cw(r) else s), N_DEV)

        def c_fin(r):
            return lax.rem(my + (1 if cw(r) else N_DEV - 1), N_DEV)

        def c_ag(r, k):
            return lax.rem(my + (N_DEV - k if cw(r) else k), N_DEV)

        def store_out(c, r, acc):
            out_ref[pl.ds(c * ch, ch), r * qn:(r + 1) * qn] = (
                jnp.maximum(acc.astype(jnp.float32) * scale, 0.0))

        def signal_credit(r):
            pl.semaphore_signal(
                credit_sems.at[r], inc=1,
                device_id=(recv_peer(r),),
                device_id_type=pl.DeviceIdType.MESH,
            )

        def mk(r, slot):
            return pltpu.make_async_remote_copy(
                src_ref=send_bufs.at[r], dst_ref=recv_bufs.at[r, slot],
                send_sem=send_sems.at[r], recv_sem=recv_sems.at[r, slot],
                device_id=(send_peer(r),),
                device_id_type=pl.DeviceIdType.MESH,
            )

        barrier = pltpu.get_barrier_semaphore()
        for nbr in (left, right):
            pl.semaphore_signal(
                barrier, inc=1,
                device_id=(nbr,), device_id_type=pl.DeviceIdType.MESH,
            )
        pl.semaphore_wait(barrier, 2)

        rings = range(N_RING)


        for r in rings:
            send_bufs[r, :, :] = pc(c_rs(r, 0), r).astype(jnp.bfloat16)
        for r in rings:
            mk(r, 0).start()

        for s in range(1, N_DEV - 1):
            for r in rings:
                mk(r, (s - 1) % 2).wait_recv()
                mk(r, 0).wait_send()
                send_bufs[r, :, :] = (
                    recv_bufs[r, (s - 1) % 2, :, :].astype(jnp.float32)
                    + pc(c_rs(r, s), r)).astype(jnp.bfloat16)
                signal_credit(r)
                if s >= 2:
                    pl.semaphore_wait(credit_sems.at[r], 1)
                mk(r, s % 2).start()

        for r in rings:
            mk(r, 0).wait_recv()
            mk(r, 0).wait_send()
            acc = recv_bufs[r, 0, :, :].astype(jnp.float32) + pc(c_fin(r), r)
            signal_credit(r)
            send_bufs[r, :, :] = acc.astype(jnp.bfloat16)
            pl.semaphore_wait(credit_sems.at[r], 1)
            mk(r, 1).start()
            store_out(c_fin(r), r, acc)

        for t in range(1, N_DEV - 1):
            k = N_DEV - 1 + t
            for r in rings:
                mk(r, (k - 1) % 2).wait_recv()
                mk(r, 0).wait_send()
                send_bufs[r, :, :] = recv_bufs[r, (k - 1) % 2, :, :]
                if t <= N_DEV - 3:
                    signal_credit(r)
                pl.semaphore_wait(credit_sems.at[r], 1)
                mk(r, k % 2).start()
                store_out(c_ag(r, t - 1), r, send_bufs[r, :, :])

        for r in rings:
            mk(r, 1).wait_recv()
            mk(r, 0).wait_send()
            store_out(c_ag(r, N_DEV - 2), r, recv_bufs[r, 1, :, :])

        @functools.partial(pl.run_scoped, sem=pltpu.SemaphoreType.REGULAR)
        def _(sem):
            for nbr in (left, right):
                pl.semaphore_signal(
                    sem, inc=1,
                    device_id=(nbr,), device_id_type=pl.DeviceIdType.MESH,
                )
            pl.semaphore_wait(sem, 2)

    return pl.pallas_call(
        body,
        out_shape=jax.ShapeDtypeStruct((m, n), jnp.float32),
        in_specs=[
            pl.BlockSpec(memory_space=pltpu.VMEM),
            pl.BlockSpec(memory_space=pltpu.VMEM),
            pl.BlockSpec(memory_space=pltpu.SMEM),
            pl.BlockSpec(memory_space=pltpu.SMEM),
        ],
        out_specs=pl.BlockSpec(memory_space=pltpu.VMEM),
        scratch_shapes=[
            pltpu.VMEM((m, k_local), jnp.bfloat16),
            pltpu.VMEM((k_local, n), jnp.bfloat16),
            pltpu.VMEM((N_RING, ch, qn), jnp.bfloat16),
            pltpu.VMEM((N_RING, 2, ch, qn), jnp.bfloat16),
            pltpu.SemaphoreType.DMA((N_RING,)),
            pltpu.SemaphoreType.DMA((N_RING, 2)),
            pltpu.SemaphoreType.REGULAR((N_RING,)),
        ],
        compiler_params=pltpu.CompilerParams(
            collective_id=0,
            vmem_limit_bytes=64 * 1024 * 1024,
        ),
    )(x, w_mat, scale_x, scale_w)


# device time: 205734 ns/iter; 3.5481x vs baseline; 1.0037x over previous
import functools

import jax
import jax.numpy as jnp
from jax import lax
from jax.experimental import pallas as pl
from jax.experimental.pallas import tpu as pltpu

N_DEV = 8
N_RING = 4


def kernel(x, w_mat, scale_x, scale_w):
    m, k_local = x.shape
    _, n = w_mat.shape
    ch = m // N_DEV
    qn = n // N_RING

    def body(x_ref, w_ref, sx_ref, sw_ref, out_ref,
             xb_ref, wb_ref, send_bufs, recv_bufs, pc_bufs,
             send_sems, recv_sems, credit_sems):
        my = lax.axis_index("i")
        left = lax.rem(my + N_DEV - 1, N_DEV)
        right = lax.rem(my + 1, N_DEV)

        scale = sx_ref[0] * sw_ref[0]

        xb_ref[:, :] = x_ref[:, :].astype(jnp.bfloat16)
        wb_ref[:, :] = w_ref[:, :].astype(jnp.bfloat16)

        cw = lambda r: r < 2
        send_peer = lambda r: right if cw(r) else left
        recv_peer = lambda r: left if cw(r) else right

        def pc(c, r):
            xs = xb_ref[pl.ds(c * ch, ch), :]
            return jax.lax.dot_general(
                xs, wb_ref[:, r * qn:(r + 1) * qn],
                dimension_numbers=(((1,), (0,)), ((), ())),
                preferred_element_type=jnp.float32,
            )

        def c_rs(r, s):
            return lax.rem(my + (N_DEV - s if cw(r) else s), N_DEV)

        def c_fin(r):
            return lax.rem(my + (1 if cw(r) else N_DEV - 1), N_DEV)

        def c_ag(r, k):
            return lax.rem(my + (N_DEV - k if cw(r) else k), N_DEV)

        def store_out(c, r, acc):
            out_ref[pl.ds(c * ch, ch), r * qn:(r + 1) * qn] = (
                jnp.maximum(acc.astype(jnp.float32) * scale, 0.0))

        def signal_credit(r):
            pl.semaphore_signal(
                credit_sems.at[r], inc=1,
                device_id=(recv_peer(r),),
                device_id_type=pl.DeviceIdType.MESH,
            )

        def mk(r, slot):
            return pltpu.make_async_remote_copy(
                src_ref=send_bufs.at[r], dst_ref=recv_bufs.at[r, slot],
                send_sem=send_sems.at[r], recv_sem=recv_sems.at[r, slot],
                device_id=(send_peer(r),),
                device_id_type=pl.DeviceIdType.MESH,
            )

        barrier = pltpu.get_barrier_semaphore()
        for nbr in (left, right):
            pl.semaphore_signal(
                barrier, inc=1,
                device_id=(nbr,), device_id_type=pl.DeviceIdType.MESH,
            )
        pl.semaphore_wait(barrier, 2)

        rings = range(N_RING)


        for r in rings:
            send_bufs[r, :, :] = pc(c_rs(r, 0), r).astype(jnp.bfloat16)
        for r in rings:
            mk(r, 0).start()

        for s in range(1, N_DEV - 1):
            for r in rings:
                pc_bufs[r, :, :] = pc(c_rs(r, s), r)
            for r in rings:
                mk(r, (s - 1) % 2).wait_recv()
                mk(r, 0).wait_send()
                send_bufs[r, :, :] = (
                    recv_bufs[r, (s - 1) % 2, :, :].astype(jnp.float32)
                    + pc_bufs[r, :, :]).astype(jnp.bfloat16)
                signal_credit(r)
                if s >= 2:
                    pl.semaphore_wait(credit_sems.at[r], 1)
                mk(r, s % 2).start()

        for r in rings:
            pc_bufs[r, :, :] = pc(c_fin(r), r)
        for r in rings:
            mk(r, 0).wait_recv()
            mk(r, 0).wait_send()
            acc = recv_bufs[r, 0, :, :].astype(jnp.float32) + pc_bufs[r, :, :]
            signal_credit(r)
            send_bufs[r, :, :] = acc.astype(jnp.bfloat16)
            pl.semaphore_wait(credit_sems.at[r], 1)
            mk(r, 1).start()
            store_out(c_fin(r), r, acc)

        for t in range(1, N_DEV - 1):
            k = N_DEV - 1 + t
            for r in rings:
                mk(r, (k - 1) % 2).wait_recv()
                mk(r, 0).wait_send()
                send_bufs[r, :, :] = recv_bufs[r, (k - 1) % 2, :, :]
                if t <= N_DEV - 3:
                    signal_credit(r)
                pl.semaphore_wait(credit_sems.at[r], 1)
                mk(r, k % 2).start()
                store_out(c_ag(r, t - 1), r, send_bufs[r, :, :])

        for r in rings:
            mk(r, 1).wait_recv()
            mk(r, 0).wait_send()
            store_out(c_ag(r, N_DEV - 2), r, recv_bufs[r, 1, :, :])

        @functools.partial(pl.run_scoped, sem=pltpu.SemaphoreType.REGULAR)
        def _(sem):
            for nbr in (left, right):
                pl.semaphore_signal(
                    sem, inc=1,
                    device_id=(nbr,), device_id_type=pl.DeviceIdType.MESH,
                )
            pl.semaphore_wait(sem, 2)

    return pl.pallas_call(
        body,
        out_shape=jax.ShapeDtypeStruct((m, n), jnp.float32),
        in_specs=[
            pl.BlockSpec(memory_space=pltpu.VMEM),
            pl.BlockSpec(memory_space=pltpu.VMEM),
            pl.BlockSpec(memory_space=pltpu.SMEM),
            pl.BlockSpec(memory_space=pltpu.SMEM),
        ],
        out_specs=pl.BlockSpec(memory_space=pltpu.VMEM),
        scratch_shapes=[
            pltpu.VMEM((m, k_local), jnp.bfloat16),
            pltpu.VMEM((k_local, n), jnp.bfloat16),
            pltpu.VMEM((N_RING, ch, qn), jnp.bfloat16),
            pltpu.VMEM((N_RING, 2, ch, qn), jnp.bfloat16),
            pltpu.VMEM((N_RING, ch, qn), jnp.float32),
            pltpu.SemaphoreType.DMA((N_RING,)),
            pltpu.SemaphoreType.DMA((N_RING, 2)),
            pltpu.SemaphoreType.REGULAR((N_RING,)),
        ],
        compiler_params=pltpu.CompilerParams(
            collective_id=0,
            vmem_limit_bytes=64 * 1024 * 1024,
        ),
    )(x, w_mat, scale_x, scale_w)
